# baseline (speedup 1.0000x reference)
import functools

import jax
import jax.numpy as jnp
from jax import lax
from jax.experimental import pallas as pl
from jax.experimental.pallas import tpu as pltpu
from jax.experimental.pallas import tpu_sc as plsc

_NI = 2
_NO = 4


@functools.cache
def _make_gather(batch, seq, V, D):
    info = plsc.get_sparse_core_info()
    L = info.num_lanes
    num_workers = info.num_cores * info.num_subcores
    n_rounds = D // num_workers
    unroll = 16
    irows = batch // 128
    assert n_rounds * num_workers == D
    assert batch % (L * unroll) == 0 and seq % (_NI * _NO) == 0

    mesh = plsc.VectorSubcoreMesh(core_axis_name="c", subcore_axis_name="s")

    @functools.partial(
        pl.kernel,
        mesh=mesh,
        out_type=jax.ShapeDtypeStruct((seq, D, batch), jnp.float32),
        scratch_types=[
            pltpu.VMEM((1, V), jnp.float32),
            *[pltpu.VMEM((irows, 128), jnp.int32) for _ in range(_NI)],
            *[pltpu.VMEM((1, batch), jnp.float32) for _ in range(_NO)],
            *[pltpu.SemaphoreType.DMA for _ in range(_NI + _NO)],
        ],
        compiler_params=pltpu.CompilerParams(
            use_tc_tiling_on_sc=True, needs_layout_passes=False
        ),
    )
    def gather_kernel(idx_hbm, table_hbm, out_hbm, trow, *bufs):
        ibuf = bufs[:_NI]
        obuf = bufs[_NI : _NI + _NO]
        isem = bufs[_NI + _NO : 2 * _NI + _NO]
        osem = bufs[2 * _NI + _NO :]
        wid = lax.axis_index("s") * info.num_cores + lax.axis_index("c")
        z16 = jnp.zeros((L,), jnp.int32)

        def idx_load(p, s):
            pltpu.async_copy(idx_hbm.at[pl.ds(s * irows, irows)], ibuf[p], isem[p])

        def idx_wait(p):
            pltpu.make_async_copy(
                idx_hbm.at[pl.ds(0, irows)], ibuf[p], isem[p]
            ).wait()

        def out_store(p, s, d):
            pltpu.async_copy(obuf[p], out_hbm.at[s, pl.ds(d, 1)], osem[p])

        def out_wait(p):
            pltpu.make_async_copy(obuf[p], out_hbm.at[0, pl.ds(0, 1)], osem[p]).wait()

        def run_round(r, carry):
            d = r * num_workers + wid
            pltpu.sync_copy(table_hbm.at[pl.ds(d, 1)], trow)
            for p in range(_NI):
                idx_load(p, p)

            def do_row(ip, op, s, first_use):
                idx_wait(ip)

                @pl.when(jnp.logical_not(first_use))
                def _():
                    out_wait(op)

                @plsc.parallel_loop(0, batch, L, unroll=unroll)
                def _inner(off):
                    v = ibuf[ip][off // 128, pl.ds(lax.rem(off, 128), L)]
                    obuf[op][0, pl.ds(off, L)] = plsc.load_gather(trow, [z16, v])

                out_store(op, s, d)

                @pl.when(s + _NI < seq)
                def _():
                    idx_load(ip, s + _NI)

            block = _NI * _NO

            def blk(g, c):
                for j in range(block):
                    first = (g == 0) if j < _NO else jnp.bool_(False)
                    do_row(j % _NI, j % _NO, g * block + j, first)
                return c

            lax.fori_loop(0, seq // block, blk, 0)
            for p in range(_NO):
                out_wait(p)
            return carry

        lax.fori_loop(0, n_rounds, run_round, 0)

    return gather_kernel


def kernel(card_indices, table):
    batch, seq = card_indices.shape
    vocab, dim = table.shape
    idx2 = card_indices.astype(jnp.int32).T.reshape(-1, 128)
    table_t = table.T
    gather = _make_gather(batch, seq, vocab, dim)
    out_t = gather(idx2, table_t)
    return jnp.transpose(out_t, (2, 0, 1))


# speedup vs baseline: 1.5978x; 1.5978x over previous
import functools

import jax
import jax.numpy as jnp
from jax import lax
from jax.experimental import pallas as pl
from jax.experimental.pallas import tpu as pltpu
from jax.experimental.pallas import tpu_sc as plsc

_NI = 2
_NO = 4
_SLAB = 8


@functools.cache
def _make_gather(batch, seq, V, D):
    info = plsc.get_sparse_core_info()
    L = info.num_lanes
    num_workers = info.num_cores * info.num_subcores
    n_rounds = D // num_workers
    unroll = 16
    irows = batch // 128
    n_slabs = seq // _SLAB
    slab_irows = _SLAB * irows
    blocks_per_slab = _SLAB // (_NI * _NO)
    assert n_rounds * num_workers == D
    assert batch % (L * unroll) == 0
    assert n_slabs * _SLAB == seq and blocks_per_slab * _NI * _NO == _SLAB

    mesh = plsc.VectorSubcoreMesh(core_axis_name="c", subcore_axis_name="s")

    @functools.partial(
        pl.kernel,
        mesh=mesh,
        out_type=jax.ShapeDtypeStruct((seq, D, batch), jnp.float32),
        scratch_types=[
            pltpu.VMEM((1, V), jnp.float32),
            *[pltpu.VMEM((irows, 128), jnp.int32) for _ in range(_NI)],
            *[pltpu.VMEM((1, batch), jnp.float32) for _ in range(_NO)],
            pltpu.VMEM_SHARED((2, slab_irows, 128), jnp.int32),
            *[pltpu.SemaphoreType.DMA for _ in range(_NI + _NO + 1)],
        ],
        compiler_params=pltpu.CompilerParams(
            use_tc_tiling_on_sc=True, needs_layout_passes=False
        ),
    )
    def gather_kernel(idx_hbm, table_hbm, out_hbm, trow, *bufs):
        ibuf = bufs[:_NI]
        obuf = bufs[_NI : _NI + _NO]
        sbuf = bufs[_NI + _NO]
        isem = bufs[_NI + _NO + 1 : 2 * _NI + _NO + 1]
        osem = bufs[2 * _NI + _NO + 1 : 2 * _NI + 2 * _NO + 1]
        psem = bufs[2 * _NI + 2 * _NO + 1]
        cid = lax.axis_index("c")
        sid = lax.axis_index("s")
        wid = sid * info.num_cores + cid
        producer = sid == 0
        z16 = jnp.zeros((L,), jnp.int32)

        def slab_load(k):
            pltpu.async_copy(
                idx_hbm.at[pl.ds(k * slab_irows, slab_irows)],
                sbuf.at[lax.rem(k, 2)],
                psem,
            )

        def slab_wait():
            pltpu.make_async_copy(
                idx_hbm.at[pl.ds(0, slab_irows)], sbuf.at[0], psem
            ).wait()

        def idx_load(p, k, w):
            pltpu.async_copy(
                sbuf.at[lax.rem(k, 2), pl.ds(w * irows, irows)], ibuf[p], isem[p]
            )

        def idx_wait(p):
            pltpu.make_async_copy(
                sbuf.at[0, pl.ds(0, irows)], ibuf[p], isem[p]
            ).wait()

        def out_store(p, s, d):
            pltpu.async_copy(obuf[p], out_hbm.at[s, pl.ds(d, 1)], osem[p])

        def out_wait(p):
            pltpu.make_async_copy(obuf[p], out_hbm.at[0, pl.ds(0, 1)], osem[p]).wait()

        def run_round(r, carry):
            d = r * num_workers + wid
            pltpu.sync_copy(table_hbm.at[pl.ds(d, 1)], trow)

            @pl.when(producer)
            def _():
                slab_load(0)

            def do_row(ip, op, k, w, first_use):
                idx_wait(ip)

                @pl.when(jnp.logical_not(first_use))
                def _():
                    out_wait(op)

                @plsc.parallel_loop(0, batch, L, unroll=unroll)
                def _inner(off):
                    v = ibuf[ip][off // 128, pl.ds(lax.rem(off, 128), L)]
                    obuf[op][0, pl.ds(off, L)] = plsc.load_gather(trow, [z16, v])

                out_store(op, k * _SLAB + w, d)

                @pl.when(w + _NI < _SLAB)
                def _():
                    idx_load(ip, k, w + _NI)

            def slab_loop(k, c):
                @pl.when(producer)
                def _():
                    slab_wait()

                plsc.subcore_barrier()

                @pl.when(jnp.logical_and(producer, k + 1 < n_slabs))
                def _():
                    slab_load(k + 1)

                for p in range(_NI):
                    idx_load(p, k, p)

                def blk(g, c2):
                    for j in range(_NI * _NO):
                        w = g * _NI * _NO + j
                        if j < _NO:
                            first = jnp.logical_and(k == 0, g == 0)
                        else:
                            first = jnp.bool_(False)
                        do_row(j % _NI, j % _NO, k, w, first)
                    return c2

                lax.fori_loop(0, blocks_per_slab, blk, 0)
                return c

            lax.fori_loop(0, n_slabs, slab_loop, 0)
            for p in range(_NO):
                out_wait(p)
            return carry

        lax.fori_loop(0, n_rounds, run_round, 0)

    return gather_kernel


def kernel(card_indices, table):
    batch, seq = card_indices.shape
    vocab, dim = table.shape
    idx2 = card_indices.astype(jnp.int32).T.reshape(-1, 128)
    table_t = table.T
    gather = _make_gather(batch, seq, vocab, dim)
    out_t = gather(idx2, table_t)
    return jnp.transpose(out_t, (2, 0, 1))
